# initial kernel scaffold (unmeasured)
import jax
import jax.numpy as jnp
from jax import lax
from jax.experimental import pallas as pl
from jax.experimental.pallas import tpu as pltpu

N_DEV = 4
M_PER = 1024
K_PER = 1024
K = 4096
N = 8192
NC = 2048
N_CHUNKS = N // NC


def kernel(x, w_mat):
    def body(x_ref, w_ref, out_ref,
             x_bf, recv_buf, w_slots,
             w_sems, send_sems, recv_sems,
             amax_send_buf, amax_recv_buf, amax_send_sems, amax_recv_sems):
        my = lax.axis_index("i")

        barrier = pltpu.get_barrier_semaphore()
        for o in range(1, N_DEV):
            t = (my + o) % N_DEV
            pl.semaphore_signal(barrier, inc=1, device_id=(t,),
                                device_id_type=pl.DeviceIdType.MESH)
        pl.semaphore_wait(barrier, N_DEV - 1)

        for o in range(N_DEV):
            t = (my + o) % N_DEV
            x_bf[o] = x_ref[pl.ds(t * M_PER, M_PER), :].astype(jnp.bfloat16)

        def a2a_desc(o):
            t = (my + o) % N_DEV
            return pltpu.make_async_remote_copy(
                src_ref=x_bf.at[o],
                dst_ref=recv_buf.at[o - 1],
                send_sem=send_sems.at[o - 1],
                recv_sem=recv_sems.at[o - 1],
                device_id=(t,),
                device_id_type=pl.DeviceIdType.MESH,
            )

        for o in (1, 3, 2):
            a2a_desc(o).start()

        blocks = [
            (x_bf, 0, my % N_DEV, None),
            (recv_buf, 0, (my - 1) % N_DEV, 1),
            (recv_buf, 2, (my + 1) % N_DEV, 3),
            (recv_buf, 1, (my + 2) % N_DEV, 2),
        ]
        n_steps = N_DEV * N_CHUNKS

        def w_dma(i, slot):
            b = i // N_CHUNKS
            nc = i % N_CHUNKS
            s = blocks[b][2]
            return pltpu.make_async_copy(
                w_ref.at[pl.ds(s * K_PER, K_PER), pl.ds(nc * NC, NC)],
                w_slots.at[slot],
                w_sems.at[slot],
            )

        w_dma(0, 0).start()
        for i in range(n_steps):
            cur = i % 2
            if i + 1 < n_steps:
                w_dma(i + 1, (i + 1) % 2).start()
            b = i // N_CHUNKS
            nc = i % N_CHUNKS
            buf, idx, s, o_recv = blocks[b]
            if nc == 0 and o_recv is not None:
                a2a_desc(o_recv).wait_recv()
            w_dma(i, cur).wait()
            wv = w_slots[cur].astype(jnp.bfloat16)
            acc = jnp.dot(buf[idx], wv, preferred_element_type=jnp.float32)
            if b == 0:
                out_ref[:, nc * NC:(nc + 1) * NC] = acc
            else:
                out_ref[:, nc * NC:(nc + 1) * NC] += acc

        local_amax = jnp.float32(0.0)
        for nc in range(N_CHUNKS):
            y = jnp.maximum(out_ref[:, nc * NC:(nc + 1) * NC], 0.0)
            out_ref[:, nc * NC:(nc + 1) * NC] = y
            local_amax = jnp.maximum(local_amax, jnp.max(y))

        amax_send_buf[...] = jnp.full((8, 128), local_amax, jnp.float32)

        def amax_desc(o):
            t = (my + o) % N_DEV
            return pltpu.make_async_remote_copy(
                src_ref=amax_send_buf,
                dst_ref=amax_recv_buf.at[o - 1],
                send_sem=amax_send_sems.at[o - 1],
                recv_sem=amax_recv_sems.at[o - 1],
                device_id=(t,),
                device_id_type=pl.DeviceIdType.MESH,
            )

        for o in (1, 2, 3):
            amax_desc(o).start()
        for o in (1, 2, 3):
            amax_desc(o).wait_recv()

        g_amax = jnp.maximum(local_amax, jnp.max(amax_recv_buf[...]))
        scale = g_amax / 127.0

        for nc in range(N_CHUNKS):
            y = out_ref[:, nc * NC:(nc + 1) * NC]
            q = jnp.clip(jnp.round(y / scale), -127.0, 127.0)
            out_ref[:, nc * NC:(nc + 1) * NC] = q * scale

        for o in (1, 2, 3):
            a2a_desc(o).wait_send()
            amax_desc(o).wait_send()

    return pl.pallas_call(
        body,
        out_shape=jax.ShapeDtypeStruct((M_PER, N), jnp.float32),
        in_specs=[
            pl.BlockSpec(memory_space=pltpu.VMEM),
            pl.BlockSpec(memory_space=pltpu.ANY),
        ],
        out_specs=pl.BlockSpec(memory_space=pltpu.VMEM),
        scratch_shapes=[
            pltpu.VMEM((N_DEV, M_PER, K_PER), jnp.bfloat16),
            pltpu.VMEM((N_DEV - 1, M_PER, K_PER), jnp.bfloat16),
            pltpu.VMEM((2, K_PER, NC), jnp.float32),
            pltpu.SemaphoreType.DMA((2,)),
            pltpu.SemaphoreType.DMA((N_DEV - 1,)),
            pltpu.SemaphoreType.DMA((N_DEV - 1,)),
            pltpu.VMEM((8, 128), jnp.float32),
            pltpu.VMEM((N_DEV - 1, 8, 128), jnp.float32),
            pltpu.SemaphoreType.DMA((N_DEV - 1,)),
            pltpu.SemaphoreType.DMA((N_DEV - 1,)),
        ],
        compiler_params=pltpu.CompilerParams(collective_id=0),
    )(x, w_mat)


# baseline (device time: 160075 ns/iter reference)
import jax
import jax.numpy as jnp
from jax import lax
from jax.experimental import pallas as pl
from jax.experimental.pallas import tpu as pltpu

N_DEV = 4
M_PER = 1024
K_PER = 1024
K = 4096
N = 8192
NC = 1024
N_CHUNKS = N // NC


def kernel(x, w_mat):
    def body(x_ref, w_ref, out_ref,
             x_bf, recv_buf, w_slots,
             w_sems, send_sems, recv_sems,
             amax_send_buf, amax_recv_buf, amax_send_sems, amax_recv_sems):
        my = lax.axis_index("i")

        barrier = pltpu.get_barrier_semaphore()
        for o in range(1, N_DEV):
            t = (my + o) % N_DEV
            pl.semaphore_signal(barrier, inc=1, device_id=(t,),
                                device_id_type=pl.DeviceIdType.MESH)
        pl.semaphore_wait(barrier, N_DEV - 1)

        def a2a_desc(o):
            t = (my + o) % N_DEV
            return pltpu.make_async_remote_copy(
                src_ref=x_bf.at[o],
                dst_ref=recv_buf.at[o - 1],
                send_sem=send_sems.at[o - 1],
                recv_sem=recv_sems.at[o - 1],
                device_id=(t,),
                device_id_type=pl.DeviceIdType.MESH,
            )

        stage_order = (2, 1, 3, 0)

        def x_stage_dma(o, slot):
            t = (my + o) % N_DEV
            return pltpu.make_async_copy(
                x_ref.at[pl.ds(t * M_PER, M_PER), :],
                w_slots.at[slot],
                w_sems.at[slot],
            )

        x_stage_dma(stage_order[0], 0).start()
        for j, o in enumerate(stage_order):
            if j + 1 < len(stage_order):
                x_stage_dma(stage_order[j + 1], (j + 1) % 2).start()
            x_stage_dma(o, j % 2).wait()
            x_bf[o] = w_slots[j % 2].astype(jnp.bfloat16)
            if o != 0:
                a2a_desc(o).start()

        blocks = [
            (x_bf, 0, my % N_DEV, None),
            (recv_buf, 0, (my - 1) % N_DEV, 1),
            (recv_buf, 2, (my + 1) % N_DEV, 3),
            (recv_buf, 1, (my + 2) % N_DEV, 2),
        ]
        n_steps = N_DEV * N_CHUNKS
        local_amax = jnp.float32(0.0)

        def w_dma(i, slot):
            b = i // N_CHUNKS
            nc = i % N_CHUNKS
            s = blocks[b][2]
            return pltpu.make_async_copy(
                w_ref.at[pl.ds(s * K_PER, K_PER), pl.ds(nc * NC, NC)],
                w_slots.at[slot],
                w_sems.at[slot],
            )

        w_dma(0, 0).start()
        for i in range(n_steps):
            cur = i % 2
            if i + 1 < n_steps:
                w_dma(i + 1, (i + 1) % 2).start()
            b = i // N_CHUNKS
            nc = i % N_CHUNKS
            buf, idx, s, o_recv = blocks[b]
            if nc == 0 and o_recv is not None:
                a2a_desc(o_recv).wait_recv()
            w_dma(i, cur).wait()
            wv = w_slots[cur].astype(jnp.bfloat16)
            acc = jnp.dot(buf[idx], wv, preferred_element_type=jnp.float32)
            col = slice(nc * NC, (nc + 1) * NC)
            if b == 0:
                out_ref[:, col] = acc
            elif b < N_DEV - 1:
                out_ref[:, col] += acc
            else:
                y = jnp.maximum(out_ref[:, col] + acc, 0.0)
                out_ref[:, col] = y
                local_amax = jnp.maximum(local_amax, jnp.max(y))

        amax_send_buf[...] = jnp.full((8, 128), local_amax, jnp.float32)

        def amax_desc(o):
            t = (my + o) % N_DEV
            return pltpu.make_async_remote_copy(
                src_ref=amax_send_buf,
                dst_ref=amax_recv_buf.at[o - 1],
                send_sem=amax_send_sems.at[o - 1],
                recv_sem=amax_recv_sems.at[o - 1],
                device_id=(t,),
                device_id_type=pl.DeviceIdType.MESH,
            )

        for o in (1, 2, 3):
            amax_desc(o).start()
        for o in (1, 2, 3):
            amax_desc(o).wait_recv()

        g_amax = jnp.maximum(local_amax, jnp.max(amax_recv_buf[...]))
        scale = g_amax / 127.0

        for nc in range(N_CHUNKS):
            col = slice(nc * NC, (nc + 1) * NC)
            q = jnp.clip(jnp.round(out_ref[:, col] / scale), -127.0, 127.0)
            out_ref[:, col] = q * scale

        for o in (1, 2, 3):
            a2a_desc(o).wait_send()
            amax_desc(o).wait_send()

    return pl.pallas_call(
        body,
        out_shape=jax.ShapeDtypeStruct((M_PER, N), jnp.float32),
        in_specs=[
            pl.BlockSpec(memory_space=pl.ANY),
            pl.BlockSpec(memory_space=pl.ANY),
        ],
        out_specs=pl.BlockSpec(memory_space=pltpu.VMEM),
        scratch_shapes=[
            pltpu.VMEM((N_DEV, M_PER, K_PER), jnp.bfloat16),
            pltpu.VMEM((N_DEV - 1, M_PER, K_PER), jnp.bfloat16),
            pltpu.VMEM((2, K_PER, NC), jnp.float32),
            pltpu.SemaphoreType.DMA((2,)),
            pltpu.SemaphoreType.DMA((N_DEV - 1,)),
            pltpu.SemaphoreType.DMA((N_DEV - 1,)),
            pltpu.VMEM((8, 128), jnp.float32),
            pltpu.VMEM((N_DEV - 1, 8, 128), jnp.float32),
            pltpu.SemaphoreType.DMA((N_DEV - 1,)),
            pltpu.SemaphoreType.DMA((N_DEV - 1,)),
        ],
        compiler_params=pltpu.CompilerParams(
            collective_id=0,
            vmem_limit_bytes=63 * 1024 * 1024,
        ),
    )(x, w_mat)


# device time: 144800 ns/iter; 1.1055x vs baseline; 1.1055x over previous
import jax

try:
    jax.config.update("jax_compilation_cache_dir", "/tmp/jax_comp_cache")
    jax.config.update("jax_persistent_cache_min_compile_time_secs", 1.0)
except Exception:
    pass

import jax.numpy as jnp
from jax import lax
from jax.experimental import pallas as pl
from jax.experimental.pallas import tpu as pltpu

N_DEV = 4
M_PER = 1024
K_PER = 1024
K = 4096
N = 8192
NC = 1024
N_CHUNKS = N // NC
NQ = 4
KQ = K_PER // NQ
SLAB = 3 * KQ
PEERS = (1, 3, 2)


def kernel(x, w_mat):
    def body(x_ref, w_ref, out_ref,
             x_bf, recv_slab, w_slots, x_stage,
             w_sems, stage_sems, send_sems, recv_sems,
             amax_send_buf, amax_recv_buf, amax_send_sems, amax_recv_sems):
        my = lax.axis_index("i")

        barrier = pltpu.get_barrier_semaphore()
        for o in range(1, N_DEV):
            t = (my + o) % N_DEV
            pl.semaphore_signal(barrier, inc=1, device_id=(t,),
                                device_id_type=pl.DeviceIdType.MESH)
        pl.semaphore_wait(barrier, N_DEV - 1)

        seg = {1: 0, 3: 1, 2: 2}

        def a2a_desc(o, q):
            t = (my + o) % N_DEV
            return pltpu.make_async_remote_copy(
                src_ref=x_bf.at[o, :, pl.ds(q * KQ, KQ)],
                dst_ref=recv_slab.at[q, :, pl.ds(seg[o] * KQ, KQ)],
                send_sem=send_sems.at[o - 1, q],
                recv_sem=recv_sems.at[o - 1, q],
                device_id=(t,),
                device_id_type=pl.DeviceIdType.MESH,
            )

        for o in (1, 3, 2, 0):
            t = (my + o) % N_DEV
            cp = pltpu.make_async_copy(
                x_ref.at[pl.ds(t * M_PER, M_PER), :],
                x_stage, stage_sems,
            )
            cp.start()
            cp.wait()
            x_bf[o] = x_stage[...].astype(jnp.bfloat16)
            if o != 0:
                a2a_desc(o, 0).start()

        local_amax = jnp.float32(0.0)

        def own_w_dma(nc, slot):
            return [pltpu.make_async_copy(
                w_ref.at[pl.ds(my * K_PER + j * 512, 512), pl.ds(nc * NC, NC)],
                w_slots.at[slot, pl.ds(j * 512, 512)],
                w_sems.at[slot, j],
            ) for j in range(2)]

        def slab_w_dma(q, nc, slot):
            cps = []
            for o in PEERS:
                s = (my - o) % N_DEV
                cps.append(pltpu.make_async_copy(
                    w_ref.at[pl.ds(s * K_PER + q * KQ, KQ), pl.ds(nc * NC, NC)],
                    w_slots.at[slot, pl.ds(seg[o] * KQ, KQ)],
                    w_sems.at[slot, seg[o]],
                ))
            return cps

        groups = [("own", None)] + [("slab", q) for q in range(NQ)]
        steps = [(kind, q, nc) for (kind, q) in groups for nc in range(N_CHUNKS)]
        n_steps = len(steps)

        def step_dmas(i, slot):
            kind, q, nc = steps[i]
            return own_w_dma(nc, slot) if kind == "own" else slab_w_dma(q, nc, slot)

        for cp in step_dmas(0, 0):
            cp.start()
        for i, (kind, q, nc) in enumerate(steps):
            if i + 1 < n_steps:
                for cp in step_dmas(i + 1, (i + 1) % 2):
                    cp.start()
            if kind == "slab" and nc == 0:
                for o in PEERS:
                    if q + 1 < NQ:
                        a2a_desc(o, q).wait_send()
                        a2a_desc(o, q + 1).start()
                for o in PEERS:
                    a2a_desc(o, q).wait_recv()
            for cp in step_dmas(i, i % 2):
                cp.wait()
            if kind == "own":
                xv = x_bf[0]
                wv = w_slots[i % 2].astype(jnp.bfloat16)
            else:
                xv = recv_slab[q]
                wv = w_slots[i % 2, 0:SLAB].astype(jnp.bfloat16)
            acc = jnp.dot(xv, wv, preferred_element_type=jnp.float32)
            col = slice(nc * NC, (nc + 1) * NC)
            if kind == "own":
                out_ref[:, col] = acc
            elif q == NQ - 1:
                y = jnp.maximum(out_ref[:, col] + acc, 0.0)
                out_ref[:, col] = y
                local_amax = jnp.maximum(local_amax, jnp.max(y))
            else:
                out_ref[:, col] += acc

        amax_send_buf[...] = jnp.full((8, 128), local_amax, jnp.float32)

        def amax_desc(o):
            t = (my + o) % N_DEV
            return pltpu.make_async_remote_copy(
                src_ref=amax_send_buf,
                dst_ref=amax_recv_buf.at[o - 1],
                send_sem=amax_send_sems.at[o - 1],
                recv_sem=amax_recv_sems.at[o - 1],
                device_id=(t,),
                device_id_type=pl.DeviceIdType.MESH,
            )

        for o in (1, 2, 3):
            amax_desc(o).start()
        for o in (1, 2, 3):
            amax_desc(o).wait_recv()

        g_amax = jnp.maximum(local_amax, jnp.max(amax_recv_buf[...]))
        scale = g_amax / 127.0

        for nc in range(N_CHUNKS):
            col = slice(nc * NC, (nc + 1) * NC)
            q8 = jnp.clip(jnp.round(out_ref[:, col] / scale), -127.0, 127.0)
            out_ref[:, col] = q8 * scale

        for o in (1, 2, 3):
            a2a_desc(o, NQ - 1).wait_send()
            amax_desc(o).wait_send()

    return pl.pallas_call(
        body,
        out_shape=jax.ShapeDtypeStruct((M_PER, N), jnp.float32),
        in_specs=[
            pl.BlockSpec(memory_space=pl.ANY),
            pl.BlockSpec(memory_space=pl.ANY),
        ],
        out_specs=pl.BlockSpec(memory_space=pltpu.VMEM),
        scratch_shapes=[
            pltpu.VMEM((N_DEV, M_PER, K_PER), jnp.bfloat16),
            pltpu.VMEM((NQ, M_PER, SLAB), jnp.bfloat16),
            pltpu.VMEM((2, K_PER, NC), jnp.float32),
            pltpu.VMEM((M_PER, K_PER), jnp.float32),
            pltpu.SemaphoreType.DMA((2, 3)),
            pltpu.SemaphoreType.DMA,
            pltpu.SemaphoreType.DMA((N_DEV - 1, NQ)),
            pltpu.SemaphoreType.DMA((N_DEV - 1, NQ)),
            pltpu.VMEM((8, 128), jnp.float32),
            pltpu.VMEM((N_DEV - 1, 8, 128), jnp.float32),
            pltpu.SemaphoreType.DMA((N_DEV - 1,)),
            pltpu.SemaphoreType.DMA((N_DEV - 1,)),
        ],
        compiler_params=pltpu.CompilerParams(
            collective_id=0,
            vmem_limit_bytes=63 * 1024 * 1024,
        ),
    )(x, w_mat)
